# baseline (speedup 1.0000x reference)
import functools

import jax
import jax.numpy as jnp
from jax import lax
from jax.experimental import pallas as pl
from jax.experimental.pallas import tpu as pltpu
from jax.experimental.pallas import tpu_sc as plsc

N = 10000
E = 320000
C = 128
K = 3
OUT = 10
L = 10
EPS = 1e-5

NC = 2
NS = 16
NW = NC * NS
G = 128
CAP = 12800
CH = CAP // G
NP = 10240
RT = NP // NW
RSC = NP // NC

_mesh = plsc.VectorSubcoreMesh(core_axis_name="c", subcore_axis_name="s")


@functools.partial(
    pl.kernel,
    out_type=jax.ShapeDtypeStruct((NP, C), jnp.float32),
    mesh=_mesh,
    scratch_types=[
        pltpu.VMEM((CH, G), jnp.int32),
        pltpu.VMEM((CH, G), jnp.int32),
        pltpu.VMEM((CH, G), jnp.float32),
        pltpu.VMEM((G, C), jnp.float32),
        pltpu.VMEM((G, C), jnp.float32),
        pltpu.VMEM((G, C), jnp.float32),
        pltpu.SemaphoreType.DMA,
        pltpu.SemaphoreType.DMA,
        pltpu.VMEM_SHARED((RSC, C), jnp.float32),
    ],
)
def _sc_matvec(h_hbm, col_hbm, rowl_hbm, w_hbm, dd_hbm,
               out_hbm, col_v, rowl_v, w_v, buf, drb, dcb,
               sem, sem2, acc):
    c = lax.axis_index("c")
    s = lax.axis_index("s")
    wid = c * NS + s

    pltpu.sync_copy(col_hbm.at[wid], col_v)
    pltpu.sync_copy(rowl_hbm.at[wid], rowl_v)
    pltpu.sync_copy(w_hbm.at[wid], w_v)

    zeros16 = jnp.zeros((16,), jnp.float32)

    def zrow(j, _):
        for r in range(C // 16):
            buf[j, pl.ds(r * 16, 16)] = zeros16
        return 0

    lax.fori_loop(0, G, zrow, 0)
    pltpu.sync_copy(buf, acc.at[pl.ds(s * RT, G)])
    pltpu.sync_copy(buf, acc.at[pl.ds(s * RT + G, G)])
    pltpu.sync_copy(buf.at[pl.ds(0, RT - 2 * G)],
                    acc.at[pl.ds(s * RT + 2 * G, RT - 2 * G)])

    def chunk(k, _):
        cp_h = pltpu.async_copy(h_hbm.at[col_v.at[k]], buf, sem)
        cp_r = pltpu.async_copy(
            dd_hbm.at[pl.ds(c * RSC, RSC)].at[rowl_v.at[k]], drb, sem2)
        cp_c = pltpu.async_copy(dd_hbm.at[col_v.at[k]], dcb, sem2)
        cp_h.wait()
        cp_r.wait()
        cp_c.wait()

        def scale16(jb, _):
            ewv = w_v[k, pl.ds(jb * 16, 16)]
            for jj in range(16):
                j = jb * 16 + jj
                ewb = jnp.full((16,), ewv[jj], dtype=jnp.float32)
                w = (drb[j, pl.ds(0, 16)] * ewb) * dcb[j, pl.ds(0, 16)]
                wn = -w
                for r in range(C // 16):
                    sl = pl.ds(r * 16, 16)
                    buf[j, sl] = buf[j, sl] * wn
            return 0

        lax.fori_loop(0, G // 16, scale16, 0)
        pltpu.sync_copy(buf, acc.at[rowl_v.at[k]], add=True)
        return 0

    lax.fori_loop(0, CH, chunk, 0)

    pltpu.sync_copy(acc.at[pl.ds(s * RT, RT)], out_hbm.at[pl.ds(wid * RT, RT)])


def _deg_finalize_body(degout_ref, o_ref):
    deg = -degout_ref[:, 0:1]
    dis = jnp.where(deg > 0, 1.0 / jnp.sqrt(deg), 0.0)
    o_ref[...] = jnp.broadcast_to(dis, (NP, C))


_deg_finalize = pl.pallas_call(
    _deg_finalize_body,
    out_shape=jax.ShapeDtypeStruct((NP, C), jnp.float32),
)


def _layer_body(h_ref, t1_ref, t2_ref, W_ref, b_ref, g_ref, be_ref, o_ref):
    h = h_ref[...]
    t1 = t1_ref[...]
    tx2 = 2.0 * t2_ref[...] - h
    z = jnp.dot(h, W_ref[0], preferred_element_type=jnp.float32)
    z = z + jnp.dot(t1, W_ref[1], preferred_element_type=jnp.float32)
    z = z + jnp.dot(tx2, W_ref[2], preferred_element_type=jnp.float32)
    z = jnp.maximum(z + b_ref[...], 0.0)
    zr = z[:N]
    mu = jnp.mean(zr, axis=0, keepdims=True)
    var = jnp.mean((zr - mu) ** 2, axis=0, keepdims=True)
    o_ref[...] = (z - mu) / jnp.sqrt(var + EPS) * g_ref[...] + be_ref[...]


_layer = pl.pallas_call(
    _layer_body,
    out_shape=jax.ShapeDtypeStruct((NP, C), jnp.float32),
)


def _final_body(h_ref, w_ref, b_ref, o_ref):
    o_ref[...] = (jnp.dot(h_ref[...], w_ref[...],
                          preferred_element_type=jnp.float32) + b_ref[...])


_final = pl.pallas_call(
    _final_body,
    out_shape=jax.ShapeDtypeStruct((NP, OUT), jnp.float32),
)


def kernel(x, edge_index, edge_weight, conv_W, conv_b, bn_gamma, bn_beta,
           lin_W, lin_b):
    row = edge_index[0]
    col = edge_index[1]

    order = jnp.argsort(row, stable=True)
    rs = row[order]
    cs = col[order]
    es = edge_weight[order]

    tids = jnp.arange(NW, dtype=jnp.int32)
    start = jnp.searchsorted(rs, tids * RT).astype(jnp.int32)
    end = jnp.searchsorted(rs, (tids + 1) * RT).astype(jnp.int32)

    pos = start[:, None] + jnp.arange(CAP, dtype=jnp.int32)[None, :]
    valid = pos < end[:, None]
    idx = jnp.minimum(pos, E - 1)
    colp = jnp.where(valid, cs[idx], 0)
    ewp = jnp.where(valid, es[idx], 0.0)
    rowgp = jnp.where(valid, rs[idx], (tids * RT)[:, None])
    rowlp = rowgp - (tids // NS * RSC)[:, None]

    col3 = colp.reshape(NW, CH, G)
    rowl3 = rowlp.reshape(NW, CH, G)
    ew3 = ewp.reshape(NW, CH, G)

    ones_h = jnp.ones((NP, C), jnp.float32)
    degout = _sc_matvec(ones_h, col3, rowl3, ew3, ones_h)
    dd = _deg_finalize(degout)

    h = jnp.concatenate([x, jnp.zeros((NP - N, C), jnp.float32)])
    for l in range(L):
        t1 = _sc_matvec(h, col3, rowl3, ew3, dd)
        t2 = _sc_matvec(t1, col3, rowl3, ew3, dd)
        h = _layer(h, t1, t2, conv_W[l], conv_b[l], bn_gamma[l], bn_beta[l])
    return _final(h, lin_W, lin_b)[:N]
